# baseline (speedup 1.0000x reference)
import jax
import jax.numpy as jnp
from jax import lax
from jax.experimental import pallas as pl
from jax.experimental.pallas import tpu as pltpu
from jax.experimental.pallas import tpu_sc as plsc

_CH = 16
_HALF = 128
_NCHUNK = _HALF // _CH


def _qpool_body(rp_hbm, out_hbm,
                rbuf0, rbuf1, cbuf0, cbuf1, dbuf0, dbuf1, obuf0, obuf1,
                semr0, semr1, semc0, semc1, semd0, semd1, semo0, semo1):
    cid = lax.axis_index("c")
    sid = lax.axis_index("s")
    lanes = lax.iota(jnp.int32, 16)

    ins = [(rbuf0, cbuf0, dbuf0, semr0, semc0, semd0),
           (rbuf1, cbuf1, dbuf1, semr1, semc1, semd1)]
    outs = [(obuf0, semo0), (obuf1, semo1)]

    def issue(k):
        rbuf, cbuf, dbuf, semr, semc, semd = ins[k % 2]
        i0 = 8 * cid + k
        rp0 = sid * 512 + 32 * i0
        return (
            pltpu.async_copy(rp_hbm.at[pl.ds(rp0, 16)], rbuf, semr),
            pltpu.async_copy(
                rp_hbm.at[pl.ds(rp0 + 16, 16), pl.ds(64 * i0 + 32, 32)],
                cbuf, semc),
            pltpu.async_copy(
                rp_hbm.at[pl.ds(rp0 + 16, 16),
                          pl.ds(1024 + 64 * i0 + 32, 32)],
                dbuf, semd),
        )

    pend_in = {0: issue(0)}
    pend_out = {}
    for k in range(_NCHUNK):
        if k + 1 < _NCHUNK:
            pend_in[k + 1] = issue(k + 1)
        for c in pend_in.pop(k):
            c.wait()
        rbuf, cbuf, dbuf, _, _, _ = ins[k % 2]
        obuf, semo = outs[k % 2]
        if k >= 2:
            pend_out.pop(k - 2).wait()
        i0 = 8 * cid + k
        base16 = 16 * i0

        for t in range(_CH):
            tf = jnp.full((16,), t, jnp.int32)

            for lp in range(16):
                av = plsc.load_gather(rbuf, [tf, 64 * lp + 2 * lanes])
                obuf[t, pl.ds(16 * lp, 16)] = av

            bv = plsc.load_gather(rbuf, [tf, 1024 + 64 * lanes + 2 * t + 1])
            plsc.addupdate_scatter(obuf, [tf, 16 * lanes + t], bv)

            cv = plsc.load_gather(cbuf, [tf, 2 * lanes])
            dv = plsc.load_gather(dbuf, [tf, jnp.full((16,), 2 * t + 1,
                                                      jnp.int32)])
            cd = cv + jnp.where(lanes == t, dv, jnp.float32(0))
            plsc.addupdate_scatter(obuf, [tf, base16 + lanes], cd)

        orow = sid * 256 + base16
        pend_out[k] = pltpu.async_copy(obuf, out_hbm.at[pl.ds(orow, _CH)],
                                       semo)
    pend_out.pop(_NCHUNK - 2).wait()
    pend_out.pop(_NCHUNK - 1).wait()


def kernel(rho, mask_x, mask_y, new_x, new_y):
    b = rho.shape[0]
    rp = rho.reshape(b * 512, 2048)

    f = pl.kernel(
        _qpool_body,
        out_type=jax.ShapeDtypeStruct((b * 256, 256), jnp.float32),
        mesh=plsc.VectorSubcoreMesh(core_axis_name="c", subcore_axis_name="s"),
        scratch_types=(
            [pltpu.VMEM((_CH, 2048), jnp.float32)] * 2
            + [pltpu.VMEM((_CH, 32), jnp.float32)] * 4
            + [pltpu.VMEM((_CH, 256), jnp.float32)] * 2
            + [pltpu.SemaphoreType.DMA] * 8
        ),
        compiler_params=pltpu.CompilerParams(use_tc_tiling_on_sc=False,
                                             needs_layout_passes=False),
    )
    out = f(rp)
    return out.reshape(b, 256, 256)
